# baseline (speedup 1.0000x reference)
import functools

import jax
import jax.numpy as jnp
from jax import lax
from jax.experimental import pallas as pl
from jax.experimental.pallas import tpu as pltpu
from jax.experimental.pallas import tpu_sc as plsc

N = 10000
E = 320000
H = 128
NC = 2
NS = 16
NW = NC * NS
EW = E // NW
CH = 80
NCH = EW // CH
RB = 80
NRB = N // RB

def _row_blocks(s, fn):
    for j in range(8):
        blk = j * NS + s
        if j * NS + NS - 1 < NRB:
            fn(pl.multiple_of(blk * RB, 8))
        else:
            @pl.when(blk < NRB)
            def _():
                fn(pl.multiple_of(blk * RB, 8))


def _deg_body(dst_hbm, out_hbm, didx_v, hist_v, dsem):
    c = lax.axis_index("c")
    s = lax.axis_index("s")
    wid = c * NS + s
    zero16 = jnp.zeros((16,), jnp.float32)
    idx_src = dst_hbm.at[pl.ds(pl.multiple_of(wid * EW, 8), EW)]
    pltpu.async_copy(idx_src, didx_v, dsem)

    def zstep(k, carry):
        hist_v[pl.ds(pl.multiple_of(k * 16, 8), 16)] = zero16
        return carry

    lax.fori_loop(0, N // 16, zstep, 0)
    pltpu.make_async_copy(idx_src, didx_v, dsem).wait()
    ones16 = jnp.ones((16,), jnp.float32)

    def step(k, carry):
        iv = didx_v[pl.ds(pl.multiple_of(k * 16, 8), 16)]
        plsc.addupdate_scatter(hist_v, [iv], ones16)
        return carry

    lax.fori_loop(0, EW // 16, step, 0)
    pltpu.sync_copy(hist_v, out_hbm.at[pl.ds(pl.multiple_of(wid * N, 8), N)])


def _agg_body(y_hbm, src_hbm, dst3_hbm, zb_hbm, out_hbm,
              srcs_v, dsts_v, rows0_v, rows1_v, acc_sh, sem0, sem1):
    c = lax.axis_index("c")
    s = lax.axis_index("s")
    wid = c * NS + s
    pltpu.sync_copy(src_hbm.at[pl.ds(pl.multiple_of(wid * EW, 8), EW)], srcs_v)
    pltpu.sync_copy(dst3_hbm.at[wid], dsts_v)

    def sidx(j):
        return srcs_v.at[pl.ds(j * CH, CH)]
    stage = rows0_v.at[pl.ds(0, RB), :]
    pltpu.sync_copy(zb_hbm, stage)
    _row_blocks(s, lambda r0: pltpu.sync_copy(stage, acc_sh.at[pl.ds(r0, RB), :]))
    plsc.subcore_barrier()

    pltpu.async_copy(y_hbm.at[sidx(0)], rows0_v, sem0)

    def pair(g, carry):
        j0 = g * 2
        j1 = j0 + 1
        pltpu.async_copy(y_hbm.at[sidx(j1)], rows1_v, sem1)
        pltpu.make_async_copy(y_hbm.at[sidx(j0)], rows0_v, sem0).wait()
        pltpu.sync_copy(rows0_v, acc_sh.at[dsts_v.at[j0]], add=True)
        pltpu.async_copy(y_hbm.at[sidx(j0 + 2)], rows0_v, sem0)
        pltpu.make_async_copy(y_hbm.at[sidx(j1)], rows1_v, sem1).wait()
        pltpu.sync_copy(rows1_v, acc_sh.at[dsts_v.at[j1]], add=True)
        return carry

    lax.fori_loop(0, NCH // 2, pair, 0)
    jl = NCH - 1
    pltpu.make_async_copy(y_hbm.at[sidx(jl)], rows0_v, sem0).wait()
    pltpu.sync_copy(rows0_v, acc_sh.at[dsts_v.at[jl]], add=True)

    plsc.subcore_barrier()

    def writeout(r0):
        sl = pl.ds(r0, RB)
        pltpu.sync_copy(acc_sh.at[sl, :], stage)
        pltpu.sync_copy(stage, out_hbm.at[c, sl, :])

    _row_blocks(s, writeout)


@functools.lru_cache(maxsize=None)
def _sc_kernels():
    mesh = plsc.VectorSubcoreMesh(
        core_axis_name="c", subcore_axis_name="s",
        num_cores=NC, num_subcores=NS)
    deg_kernel = pl.kernel(
        _deg_body,
        out_type=jax.ShapeDtypeStruct((NW * N,), jnp.float32),
        mesh=mesh,
        compiler_params=pltpu.CompilerParams(needs_layout_passes=False),
        scratch_types=[
            pltpu.VMEM((EW,), jnp.int32),
            pltpu.VMEM((N,), jnp.float32),
            pltpu.SemaphoreType.DMA,
        ],
    )
    agg_kernel = pl.kernel(
        _agg_body,
        out_type=jax.ShapeDtypeStruct((NC, N, H), jnp.float32),
        mesh=mesh,
        scratch_types=[
            pltpu.VMEM((EW,), jnp.int32),
            pltpu.VMEM((NCH, CH), jnp.int32),
            pltpu.VMEM((CH, H), jnp.float32),
            pltpu.VMEM((CH, H), jnp.float32),
            pltpu.VMEM_SHARED((N, H), jnp.float32),
            pltpu.SemaphoreType.DMA,
            pltpu.SemaphoreType.DMA,
        ],
    )
    return deg_kernel, agg_kernel


BLK = 1000
NBLK = N // BLK


def _dis_from(deg_ref):
    d = jnp.sum(deg_ref[...], axis=0) + 1.0
    return lax.rsqrt(d)


def _mm_scale_body(x_ref, w_ref, deg_ref, o_ref):
    dis = _dis_from(deg_ref)
    o_ref[...] = jnp.dot(x_ref[...], w_ref[...],
                         preferred_element_type=jnp.float32, precision=lax.Precision.HIGHEST) * dis


def _ln_relu(srow, g_ref, be_ref):
    mu = jnp.mean(srow, axis=1, keepdims=True)
    t = srow - mu
    var = jnp.mean(t * t, axis=1, keepdims=True)
    h = t * lax.rsqrt(var + 1e-5) * g_ref[...] + be_ref[...]
    return jnp.maximum(h, 0.0)


def _mid_body(agg_ref, y_ref, deg_ref, b_ref, g_ref, be_ref, w2_ref, o_ref):
    dis = _dis_from(deg_ref)
    srow = (agg_ref[0] + agg_ref[1] + y_ref[...]) * dis + b_ref[...]
    h = _ln_relu(srow, g_ref, be_ref)
    o_ref[...] = jnp.dot(h, w2_ref[...],
                         preferred_element_type=jnp.float32, precision=lax.Precision.HIGHEST) * dis


def _final_body(agg_ref, y_ref, deg_ref, b_ref, g_ref, be_ref, bt_ref,
                wl_ref, bl_ref, o_ref, pool_acc, cnt_acc):
    i = pl.program_id(0)
    dis = _dis_from(deg_ref)
    srow = (agg_ref[0] + agg_ref[1] + y_ref[...]) * dis + b_ref[...]
    h = _ln_relu(srow, g_ref, be_ref)
    onehot = (bt_ref[...] == lax.broadcasted_iota(jnp.int32, (1, 64), 1))
    p = onehot.astype(jnp.float32)
    dn = (((0,), (0,)), ((), ()))
    pp = lax.dot_general(p, h, dn, preferred_element_type=jnp.float32, precision=lax.Precision.HIGHEST)
    cc = lax.dot_general(p, jnp.ones_like(h), dn,
                         preferred_element_type=jnp.float32, precision=lax.Precision.HIGHEST)

    @pl.when(i == 0)
    def _():
        pool_acc[...] = pp
        cnt_acc[...] = cc

    @pl.when(i > 0)
    def _():
        pool_acc[...] += pp
        cnt_acc[...] += cc

    @pl.when(i == pl.num_programs(0) - 1)
    def _():
        pooled = pool_acc[...] / jnp.maximum(cnt_acc[...], 1.0)
        o_ref[...] = jnp.dot(pooled, wl_ref[...],
                             preferred_element_type=jnp.float32, precision=lax.Precision.HIGHEST) + bl_ref[...]


def _row_spec(last):
    return pl.BlockSpec((BLK, last), lambda i: (i, 0))


_FULL_W = pl.BlockSpec((H, H), lambda i: (0, 0))
_DEG_SPEC = pl.BlockSpec((NW, BLK, 1), lambda i: (0, i, 0))
_AGG_SPEC = pl.BlockSpec((NC, BLK, H), lambda i: (0, i, 0))
_VEC_SPEC = pl.BlockSpec((1, H), lambda i: (0, 0))

_mm_scale = pl.pallas_call(
    _mm_scale_body,
    grid=(NBLK,),
    in_specs=[_row_spec(H), _FULL_W, _DEG_SPEC],
    out_specs=_row_spec(H),
    out_shape=jax.ShapeDtypeStruct((N, H), jnp.float32),
)

_mid = pl.pallas_call(
    _mid_body,
    grid=(NBLK,),
    in_specs=[_AGG_SPEC, _row_spec(H), _DEG_SPEC,
              _VEC_SPEC, _VEC_SPEC, _VEC_SPEC, _FULL_W],
    out_specs=_row_spec(H),
    out_shape=jax.ShapeDtypeStruct((N, H), jnp.float32),
)

_final = pl.pallas_call(
    _final_body,
    grid=(NBLK,),
    in_specs=[_AGG_SPEC, _row_spec(H), _DEG_SPEC,
              _VEC_SPEC, _VEC_SPEC, _VEC_SPEC,
              pl.BlockSpec((BLK, 1), lambda i: (i, 0)),
              pl.BlockSpec((H, 10), lambda i: (0, 0)),
              pl.BlockSpec((1, 10), lambda i: (0, 0))],
    out_specs=pl.BlockSpec((64, 10), lambda i: (0, 0)),
    out_shape=jax.ShapeDtypeStruct((64, 10), jnp.float32),
    scratch_shapes=[pltpu.VMEM((64, H), jnp.float32),
                    pltpu.VMEM((64, H), jnp.float32)],
)


def kernel(x, edge_index, batch, W1, b1, g1, be1, W2, b2, g2, be2, Wl, bl):
    src1 = edge_index[0]
    dst3 = edge_index[1].reshape(NW, NCH, CH)
    zbH = jnp.zeros((RB, H), jnp.float32)
    bt = batch.reshape(N, 1)

    deg_kernel, agg_kernel = _sc_kernels()
    deg = deg_kernel(edge_index[1]).reshape(NW, N, 1)
    y1 = _mm_scale(x, W1, deg)
    agg1 = agg_kernel(y1, src1, dst3, zbH)
    y2 = _mid(agg1, y1, deg, b1.reshape(1, H), g1.reshape(1, H),
              be1.reshape(1, H), W2)
    agg2 = agg_kernel(y2, src1, dst3, zbH)
    out = _final(agg2, y2, deg, b2.reshape(1, H), g2.reshape(1, H),
                 be2.reshape(1, H), bt, Wl, bl.reshape(1, 10))
    return out


# speedup vs baseline: 1.5371x; 1.5371x over previous
import functools

import jax
import jax.numpy as jnp
from jax import lax
from jax.experimental import pallas as pl
from jax.experimental.pallas import tpu as pltpu
from jax.experimental.pallas import tpu_sc as plsc

N = 10000
E = 320000
H = 128
NC = 2
NS = 16
NW = NC * NS
EW = E // NW
CH = 80
NCH = EW // CH
RB = 80
NRB = N // RB

def _row_blocks(s, fn):
    for j in range(8):
        blk = j * NS + s
        if j * NS + NS - 1 < NRB:
            fn(pl.multiple_of(blk * RB, 8))
        else:
            @pl.when(blk < NRB)
            def _():
                fn(pl.multiple_of(blk * RB, 8))


def _deg_body(dst_hbm, out_hbm, didx_v, hist_v, dsem):
    c = lax.axis_index("c")
    s = lax.axis_index("s")
    wid = c * NS + s
    zero16 = jnp.zeros((16,), jnp.float32)
    idx_src = dst_hbm.at[pl.ds(pl.multiple_of(wid * EW, 8), EW)]
    pltpu.async_copy(idx_src, didx_v, dsem)

    def zstep(k, carry):
        hist_v[pl.ds(pl.multiple_of(k * 16, 8), 16)] = zero16
        return carry

    lax.fori_loop(0, N // 16, zstep, 0)
    pltpu.make_async_copy(idx_src, didx_v, dsem).wait()
    ones16 = jnp.ones((16,), jnp.float32)

    def step(k, carry):
        iv = didx_v[pl.ds(pl.multiple_of(k * 16, 8), 16)]
        plsc.addupdate_scatter(hist_v, [iv], ones16)
        return carry

    lax.fori_loop(0, EW // 16, step, 0)
    for blk in range(NBLK):
        pltpu.sync_copy(
            hist_v.at[pl.ds(blk * BLK, BLK)],
            out_hbm.at[pl.ds(pl.multiple_of(blk * NW * BLK + wid * BLK, 8), BLK)])


def _agg_body(y_hbm, src_hbm, dst3_hbm, zb_hbm, out_hbm,
              srcs_v, dsts_v, rows0_v, rows1_v, acc_sh, sem0, sem1):
    c = lax.axis_index("c")
    s = lax.axis_index("s")
    wid = c * NS + s
    pltpu.sync_copy(src_hbm.at[pl.ds(pl.multiple_of(wid * EW, 8), EW)], srcs_v)
    pltpu.sync_copy(dst3_hbm.at[wid], dsts_v)

    def sidx(j):
        return srcs_v.at[pl.ds(j * CH, CH)]
    stage = rows0_v.at[pl.ds(0, RB), :]
    pltpu.sync_copy(zb_hbm, stage)
    _row_blocks(s, lambda r0: pltpu.sync_copy(stage, acc_sh.at[pl.ds(r0, RB), :]))
    plsc.subcore_barrier()

    pltpu.async_copy(y_hbm.at[sidx(0)], rows0_v, sem0)

    def pair(g, carry):
        j0 = g * 2
        j1 = j0 + 1
        pltpu.async_copy(y_hbm.at[sidx(j1)], rows1_v, sem1)
        pltpu.make_async_copy(y_hbm.at[sidx(j0)], rows0_v, sem0).wait()
        pltpu.sync_copy(rows0_v, acc_sh.at[dsts_v.at[j0]], add=True)
        pltpu.async_copy(y_hbm.at[sidx(j0 + 2)], rows0_v, sem0)
        pltpu.make_async_copy(y_hbm.at[sidx(j1)], rows1_v, sem1).wait()
        pltpu.sync_copy(rows1_v, acc_sh.at[dsts_v.at[j1]], add=True)
        return carry

    lax.fori_loop(0, NCH // 2, pair, 0)
    jl = NCH - 1
    pltpu.make_async_copy(y_hbm.at[sidx(jl)], rows0_v, sem0).wait()
    pltpu.sync_copy(rows0_v, acc_sh.at[dsts_v.at[jl]], add=True)

    plsc.subcore_barrier()

    def writeout(r0):
        sl = pl.ds(r0, RB)
        pltpu.sync_copy(acc_sh.at[sl, :], stage)
        pltpu.sync_copy(stage, out_hbm.at[c, sl, :])

    _row_blocks(s, writeout)


@functools.lru_cache(maxsize=None)
def _sc_kernels():
    mesh = plsc.VectorSubcoreMesh(
        core_axis_name="c", subcore_axis_name="s",
        num_cores=NC, num_subcores=NS)
    deg_kernel = pl.kernel(
        _deg_body,
        out_type=jax.ShapeDtypeStruct((NW * N,), jnp.float32),
        mesh=mesh,
        compiler_params=pltpu.CompilerParams(needs_layout_passes=False),
        scratch_types=[
            pltpu.VMEM((EW,), jnp.int32),
            pltpu.VMEM((N,), jnp.float32),
            pltpu.SemaphoreType.DMA,
        ],
    )
    agg_kernel = pl.kernel(
        _agg_body,
        out_type=jax.ShapeDtypeStruct((NC, N, H), jnp.float32),
        mesh=mesh,
        scratch_types=[
            pltpu.VMEM((EW,), jnp.int32),
            pltpu.VMEM((NCH, CH), jnp.int32),
            pltpu.VMEM((CH, H), jnp.float32),
            pltpu.VMEM((CH, H), jnp.float32),
            pltpu.VMEM_SHARED((N, H), jnp.float32),
            pltpu.SemaphoreType.DMA,
            pltpu.SemaphoreType.DMA,
        ],
    )
    return deg_kernel, agg_kernel


BLK = 1000
NBLK = N // BLK


def _dis_from(deg_ref3):
    deg_ref = deg_ref3[0]
    ones_col = jnp.ones((NW, 1), jnp.float32)
    d = lax.dot_general(deg_ref, ones_col, (((0,), (0,)), ((), ())),
                        preferred_element_type=jnp.float32,
                        precision=lax.Precision.HIGHEST) + 1.0
    return lax.rsqrt(d)


def _mm_scale_body(x_ref, w_ref, deg_ref, o_ref):
    dis = _dis_from(deg_ref)
    o_ref[...] = jnp.dot(x_ref[...], w_ref[...],
                         preferred_element_type=jnp.float32, precision=lax.Precision.HIGHEST) * dis


def _ln_relu(srow, g_ref, be_ref):
    mu = jnp.mean(srow, axis=1, keepdims=True)
    t = srow - mu
    var = jnp.mean(t * t, axis=1, keepdims=True)
    h = t * lax.rsqrt(var + 1e-5) * g_ref[...] + be_ref[...]
    return jnp.maximum(h, 0.0)


def _mid_body(agg_ref, y_ref, deg_ref, b_ref, g_ref, be_ref, w2_ref, o_ref):
    dis = _dis_from(deg_ref)
    srow = (agg_ref[0] + agg_ref[1] + y_ref[...]) * dis + b_ref[...]
    h = _ln_relu(srow, g_ref, be_ref)
    o_ref[...] = jnp.dot(h, w2_ref[...],
                         preferred_element_type=jnp.float32, precision=lax.Precision.HIGHEST) * dis


def _final_body(agg_ref, y_ref, deg_ref, b_ref, g_ref, be_ref, bt_ref,
                wl_ref, bl_ref, o_ref, pool_acc, cnt_acc):
    i = pl.program_id(0)
    dis = _dis_from(deg_ref)
    srow = (agg_ref[0] + agg_ref[1] + y_ref[...]) * dis + b_ref[...]
    h = _ln_relu(srow, g_ref, be_ref)
    onehot = (bt_ref[...] == lax.broadcasted_iota(jnp.int32, (1, 64), 1))
    p = onehot.astype(jnp.float32)
    dn = (((0,), (0,)), ((), ()))
    pp = lax.dot_general(p, h, dn, preferred_element_type=jnp.float32, precision=lax.Precision.HIGHEST)
    cc = lax.dot_general(p, jnp.ones_like(h), dn,
                         preferred_element_type=jnp.float32, precision=lax.Precision.HIGHEST)

    @pl.when(i == 0)
    def _():
        pool_acc[...] = pp
        cnt_acc[...] = cc

    @pl.when(i > 0)
    def _():
        pool_acc[...] += pp
        cnt_acc[...] += cc

    @pl.when(i == pl.num_programs(0) - 1)
    def _():
        pooled = pool_acc[...] / jnp.maximum(cnt_acc[...], 1.0)
        o_ref[...] = jnp.dot(pooled, wl_ref[...],
                             preferred_element_type=jnp.float32, precision=lax.Precision.HIGHEST) + bl_ref[...]


def _row_spec(last):
    return pl.BlockSpec((BLK, last), lambda i: (i, 0))


_FULL_W = pl.BlockSpec((H, H), lambda i: (0, 0))
_DEG_SPEC = pl.BlockSpec((1, NW, BLK), lambda i: (i, 0, 0))
_AGG_SPEC = pl.BlockSpec((NC, BLK, H), lambda i: (0, i, 0))
_VEC_SPEC = pl.BlockSpec((1, H), lambda i: (0, 0))

_mm_scale = pl.pallas_call(
    _mm_scale_body,
    grid=(NBLK,),
    in_specs=[_row_spec(H), _FULL_W, _DEG_SPEC],
    out_specs=_row_spec(H),
    out_shape=jax.ShapeDtypeStruct((N, H), jnp.float32),
)

_mid = pl.pallas_call(
    _mid_body,
    grid=(NBLK,),
    in_specs=[_AGG_SPEC, _row_spec(H), _DEG_SPEC,
              _VEC_SPEC, _VEC_SPEC, _VEC_SPEC, _FULL_W],
    out_specs=_row_spec(H),
    out_shape=jax.ShapeDtypeStruct((N, H), jnp.float32),
)

_final = pl.pallas_call(
    _final_body,
    grid=(NBLK,),
    in_specs=[_AGG_SPEC, _row_spec(H), _DEG_SPEC,
              _VEC_SPEC, _VEC_SPEC, _VEC_SPEC,
              pl.BlockSpec((BLK, 1), lambda i: (i, 0)),
              pl.BlockSpec((H, 10), lambda i: (0, 0)),
              pl.BlockSpec((1, 10), lambda i: (0, 0))],
    out_specs=pl.BlockSpec((64, 10), lambda i: (0, 0)),
    out_shape=jax.ShapeDtypeStruct((64, 10), jnp.float32),
    scratch_shapes=[pltpu.VMEM((64, H), jnp.float32),
                    pltpu.VMEM((64, H), jnp.float32)],
)


def kernel(x, edge_index, batch, W1, b1, g1, be1, W2, b2, g2, be2, Wl, bl):
    src1 = edge_index[0]
    dst3 = edge_index[1].reshape(NW, NCH, CH)
    zbH = jnp.zeros((RB, H), jnp.float32)
    bt = batch.reshape(N, 1)

    deg_kernel, agg_kernel = _sc_kernels()
    deg = deg_kernel(edge_index[1]).reshape(NBLK, NW, BLK)
    y1 = _mm_scale(x, W1, deg)
    agg1 = agg_kernel(y1, src1, dst3, zbH)
    y2 = _mid(agg1, y1, deg, b1.reshape(1, H), g1.reshape(1, H),
              be1.reshape(1, H), W2)
    agg2 = agg_kernel(y2, src1, dst3, zbH)
    out = _final(agg2, y2, deg, b2.reshape(1, H), g2.reshape(1, H),
                 be2.reshape(1, H), bt, Wl, bl.reshape(1, 10))
    return out


# speedup vs baseline: 1.7739x; 1.1541x over previous
import functools

import jax
import jax.numpy as jnp
from jax import lax
from jax.experimental import pallas as pl
from jax.experimental.pallas import tpu as pltpu
from jax.experimental.pallas import tpu_sc as plsc

N = 10000
E = 320000
H = 128
NC = 2
NS = 16
NW = NC * NS
EW = E // NW
CH = 80
NCH = EW // CH
RB = 80
NRB = N // RB

def _row_blocks(s, fn):
    for j in range(8):
        blk = j * NS + s
        if j * NS + NS - 1 < NRB:
            fn(pl.multiple_of(blk * RB, 8))
        else:
            @pl.when(blk < NRB)
            def _():
                fn(pl.multiple_of(blk * RB, 8))


def _deg_body(dst3_hbm, out_hbm, didx_v, hist_v, dsem):
    c = lax.axis_index("c")
    s = lax.axis_index("s")
    wid = c * NS + s
    zero16 = jnp.zeros((16,), jnp.float32)
    idx_src = dst3_hbm.at[wid]
    pltpu.async_copy(idx_src, didx_v, dsem)

    def zstep(k, carry):
        hist_v[pl.ds(pl.multiple_of(k * 16, 8), 16)] = zero16
        return carry

    lax.fori_loop(0, N // 16, zstep, 0)
    pltpu.make_async_copy(idx_src, didx_v, dsem).wait()
    ones16 = jnp.ones((16,), jnp.float32)

    def step(j, carry):
        for k in range(CH // 16):
            iv = didx_v[j, pl.ds(k * 16, 16)]
            plsc.addupdate_scatter(hist_v, [iv], ones16)
        return carry

    lax.fori_loop(0, NCH, step, 0)
    for blk in range(NBLK):
        pltpu.sync_copy(
            hist_v.at[pl.ds(blk * BLK, BLK)],
            out_hbm.at[pl.ds(pl.multiple_of(blk * NW * BLK + wid * BLK, 8), BLK)])


def _agg_body(y_hbm, src_hbm, dst3_hbm, zb_hbm, out_hbm,
              srcs_v, dsts_v, rows0_v, rows1_v, acc_sh, sem0, sem1):
    c = lax.axis_index("c")
    s = lax.axis_index("s")
    wid = c * NS + s
    pltpu.sync_copy(src_hbm.at[pl.ds(pl.multiple_of(wid * EW, 8), EW)], srcs_v)
    pltpu.sync_copy(dst3_hbm.at[wid], dsts_v)

    def sidx(j):
        return srcs_v.at[pl.ds(j * CH, CH)]
    stage = rows0_v.at[pl.ds(0, RB), :]
    pltpu.sync_copy(zb_hbm, stage)
    _row_blocks(s, lambda r0: pltpu.sync_copy(stage, acc_sh.at[pl.ds(r0, RB), :]))
    plsc.subcore_barrier()

    pltpu.async_copy(y_hbm.at[sidx(0)], rows0_v, sem0)

    def pair(g, carry):
        j0 = g * 2
        j1 = j0 + 1
        pltpu.async_copy(y_hbm.at[sidx(j1)], rows1_v, sem1)
        pltpu.make_async_copy(y_hbm.at[sidx(j0)], rows0_v, sem0).wait()
        pltpu.sync_copy(rows0_v, acc_sh.at[dsts_v.at[j0]], add=True)
        pltpu.async_copy(y_hbm.at[sidx(j0 + 2)], rows0_v, sem0)
        pltpu.make_async_copy(y_hbm.at[sidx(j1)], rows1_v, sem1).wait()
        pltpu.sync_copy(rows1_v, acc_sh.at[dsts_v.at[j1]], add=True)
        return carry

    lax.fori_loop(0, NCH // 2, pair, 0)
    jl = NCH - 1
    pltpu.make_async_copy(y_hbm.at[sidx(jl)], rows0_v, sem0).wait()
    pltpu.sync_copy(rows0_v, acc_sh.at[dsts_v.at[jl]], add=True)

    plsc.subcore_barrier()

    def writeout(r0):
        sl = pl.ds(r0, RB)
        pltpu.sync_copy(acc_sh.at[sl, :], stage)
        pltpu.sync_copy(stage, out_hbm.at[c, sl, :])

    _row_blocks(s, writeout)


@functools.lru_cache(maxsize=None)
def _sc_kernels():
    mesh = plsc.VectorSubcoreMesh(
        core_axis_name="c", subcore_axis_name="s",
        num_cores=NC, num_subcores=NS)
    deg_kernel = pl.kernel(
        _deg_body,
        out_type=jax.ShapeDtypeStruct((NW * N,), jnp.float32),
        mesh=mesh,
        compiler_params=pltpu.CompilerParams(needs_layout_passes=False),
        scratch_types=[
            pltpu.VMEM((NCH, CH), jnp.int32),
            pltpu.VMEM((N,), jnp.float32),
            pltpu.SemaphoreType.DMA,
        ],
    )
    agg_kernel = pl.kernel(
        _agg_body,
        out_type=jax.ShapeDtypeStruct((NC, N, H), jnp.float32),
        mesh=mesh,
        scratch_types=[
            pltpu.VMEM((EW,), jnp.int32),
            pltpu.VMEM((NCH, CH), jnp.int32),
            pltpu.VMEM((CH, H), jnp.float32),
            pltpu.VMEM((CH, H), jnp.float32),
            pltpu.VMEM_SHARED((N, H), jnp.float32),
            pltpu.SemaphoreType.DMA,
            pltpu.SemaphoreType.DMA,
        ],
    )
    return deg_kernel, agg_kernel


BLK = 1000
NBLK = N // BLK


def _dis_from(deg_ref3):
    deg_ref = deg_ref3[0]
    ones_col = jnp.ones((NW, 1), jnp.float32)
    d = lax.dot_general(deg_ref, ones_col, (((0,), (0,)), ((), ())),
                        preferred_element_type=jnp.float32,
                        precision=lax.Precision.HIGHEST) + 1.0
    return lax.rsqrt(d)


def _mm_scale_body(x_ref, w_ref, deg_ref, o_ref, disb_ref):
    dis = _dis_from(deg_ref)
    o_ref[...] = jnp.dot(x_ref[...], w_ref[...],
                         preferred_element_type=jnp.float32, precision=lax.Precision.HIGHEST) * dis
    disb_ref[...] = jnp.broadcast_to(dis, (BLK, H))


def _ln_relu(srow, g_ref, be_ref):
    mu = jnp.mean(srow, axis=1, keepdims=True)
    t = srow - mu
    var = jnp.mean(t * t, axis=1, keepdims=True)
    h = t * lax.rsqrt(var + 1e-5) * g_ref[...] + be_ref[...]
    return jnp.maximum(h, 0.0)


def _mid_body(agg_ref, y_ref, disb_ref, b_ref, g_ref, be_ref, w2_ref, o_ref):
    disb = disb_ref[...]
    srow = (agg_ref[0] + agg_ref[1] + y_ref[...]) * disb + b_ref[...]
    h = _ln_relu(srow, g_ref, be_ref)
    o_ref[...] = jnp.dot(h, w2_ref[...],
                         preferred_element_type=jnp.float32, precision=lax.Precision.HIGHEST) * disb


def _final_body(agg_ref, y_ref, disb_ref, b_ref, g_ref, be_ref, bt_ref,
                wl_ref, bl_ref, o_ref, pool_acc, cnt_acc):
    i = pl.program_id(0)
    srow = (agg_ref[0] + agg_ref[1] + y_ref[...]) * disb_ref[...] + b_ref[...]
    h = _ln_relu(srow, g_ref, be_ref)
    onehot = (bt_ref[...] == lax.broadcasted_iota(jnp.int32, (1, 64), 1))
    p = onehot.astype(jnp.float32)
    dn = (((0,), (0,)), ((), ()))
    pp = lax.dot_general(p, h, dn, preferred_element_type=jnp.float32, precision=lax.Precision.HIGHEST)
    cc = lax.dot_general(p, jnp.ones_like(h), dn,
                         preferred_element_type=jnp.float32, precision=lax.Precision.HIGHEST)

    @pl.when(i == 0)
    def _():
        pool_acc[...] = pp
        cnt_acc[...] = cc

    @pl.when(i > 0)
    def _():
        pool_acc[...] += pp
        cnt_acc[...] += cc

    @pl.when(i == pl.num_programs(0) - 1)
    def _():
        pooled = pool_acc[...] / jnp.maximum(cnt_acc[...], 1.0)
        o_ref[...] = jnp.dot(pooled, wl_ref[...],
                             preferred_element_type=jnp.float32, precision=lax.Precision.HIGHEST) + bl_ref[...]


def _row_spec(last):
    return pl.BlockSpec((BLK, last), lambda i: (i, 0))


_FULL_W = pl.BlockSpec((H, H), lambda i: (0, 0))
_DEG_SPEC = pl.BlockSpec((1, NW, BLK), lambda i: (i, 0, 0))
_AGG_SPEC = pl.BlockSpec((NC, BLK, H), lambda i: (0, i, 0))
_VEC_SPEC = pl.BlockSpec((1, H), lambda i: (0, 0))

_mm_scale = pl.pallas_call(
    _mm_scale_body,
    grid=(NBLK,),
    in_specs=[_row_spec(H), _FULL_W, _DEG_SPEC],
    out_specs=[_row_spec(H), _row_spec(H)],
    out_shape=[jax.ShapeDtypeStruct((N, H), jnp.float32),
               jax.ShapeDtypeStruct((N, H), jnp.float32)],
)

_mid = pl.pallas_call(
    _mid_body,
    grid=(NBLK,),
    in_specs=[_AGG_SPEC, _row_spec(H), _row_spec(H),
              _VEC_SPEC, _VEC_SPEC, _VEC_SPEC, _FULL_W],
    out_specs=_row_spec(H),
    out_shape=jax.ShapeDtypeStruct((N, H), jnp.float32),
)

_final = pl.pallas_call(
    _final_body,
    grid=(NBLK,),
    in_specs=[_AGG_SPEC, _row_spec(H), _row_spec(H),
              _VEC_SPEC, _VEC_SPEC, _VEC_SPEC,
              pl.BlockSpec((BLK, 1), lambda i: (i, 0)),
              pl.BlockSpec((H, 10), lambda i: (0, 0)),
              pl.BlockSpec((1, 10), lambda i: (0, 0))],
    out_specs=pl.BlockSpec((64, 10), lambda i: (0, 0)),
    out_shape=jax.ShapeDtypeStruct((64, 10), jnp.float32),
    scratch_shapes=[pltpu.VMEM((64, H), jnp.float32),
                    pltpu.VMEM((64, H), jnp.float32)],
)


def kernel(x, edge_index, batch, W1, b1, g1, be1, W2, b2, g2, be2, Wl, bl):
    src1 = edge_index[0]
    dst3 = edge_index[1].reshape(NW, NCH, CH)
    zbH = jnp.zeros((RB, H), jnp.float32)
    bt = batch.reshape(N, 1)

    deg_kernel, agg_kernel = _sc_kernels()
    deg = deg_kernel(dst3).reshape(NBLK, NW, BLK)
    y1, disb = _mm_scale(x, W1, deg)
    agg1 = agg_kernel(y1, src1, dst3, zbH)
    y2 = _mid(agg1, y1, disb, b1.reshape(1, H), g1.reshape(1, H),
              be1.reshape(1, H), W2)
    agg2 = agg_kernel(y2, src1, dst3, zbH)
    out = _final(agg2, y2, disb, b2.reshape(1, H), g2.reshape(1, H),
                 be2.reshape(1, H), bt, Wl, bl.reshape(1, 10))
    return out


# speedup vs baseline: 1.8384x; 1.0363x over previous
import functools

import jax
import jax.numpy as jnp
from jax import lax
from jax.experimental import pallas as pl
from jax.experimental.pallas import tpu as pltpu
from jax.experimental.pallas import tpu_sc as plsc

N = 10000
E = 320000
H = 128
NC = 2
NS = 16
NW = NC * NS
EW = E // NW
CH = 80
NCH = EW // CH
RB = 80
NRB = N // RB

def _row_blocks(s, fn):
    for j in range(8):
        blk = j * NS + s
        if j * NS + NS - 1 < NRB:
            fn(pl.multiple_of(blk * RB, 8))
        else:
            @pl.when(blk < NRB)
            def _():
                fn(pl.multiple_of(blk * RB, 8))


def _deg_body(dst3_hbm, out_hbm, didx_v, hist_v, dsem):
    c = lax.axis_index("c")
    s = lax.axis_index("s")
    wid = c * NS + s
    zero16 = jnp.zeros((16,), jnp.float32)
    idx_src = dst3_hbm.at[wid]
    pltpu.async_copy(idx_src, didx_v, dsem)

    def zstep(k, carry):
        hist_v[pl.ds(pl.multiple_of(k * 16, 8), 16)] = zero16
        return carry

    lax.fori_loop(0, N // 16, zstep, 0)
    pltpu.make_async_copy(idx_src, didx_v, dsem).wait()
    ones16 = jnp.ones((16,), jnp.float32)

    def step(j, carry):
        for k in range(CH // 16):
            iv = didx_v[j, pl.ds(k * 16, 16)]
            plsc.addupdate_scatter(hist_v, [iv], ones16)
        return carry

    lax.fori_loop(0, NCH, step, 0)
    for blk in range(NBLK):
        pltpu.sync_copy(
            hist_v.at[pl.ds(blk * BLK, BLK)],
            out_hbm.at[pl.ds(pl.multiple_of(blk * NW * BLK + wid * BLK, 8), BLK)])


def _agg_body(y_hbm, src_hbm, dst3_hbm, out_hbm,
              srcs_v, dsts_v, rows0_v, rows1_v, acc_sh, sem0, sem1):
    c = lax.axis_index("c")
    s = lax.axis_index("s")
    wid = c * NS + s
    src_src = src_hbm.at[pl.ds(pl.multiple_of(wid * EW, 8), EW)]
    pltpu.async_copy(src_src, srcs_v, sem0)
    pltpu.async_copy(dst3_hbm.at[wid], dsts_v, sem1)

    def sidx(j):
        return srcs_v.at[pl.ds(j * CH, CH)]
    stage = rows0_v.at[pl.ds(0, RB), :]
    zero16 = jnp.zeros((16,), jnp.float32)

    def zrow(r, carry):
        for cc in range(H // 16):
            rows0_v[r, pl.ds(cc * 16, 16)] = zero16
        return carry

    lax.fori_loop(0, RB, zrow, 0)
    _row_blocks(s, lambda r0: pltpu.sync_copy(stage, acc_sh.at[pl.ds(r0, RB), :]))
    pltpu.make_async_copy(src_src, srcs_v, sem0).wait()
    pltpu.make_async_copy(dst3_hbm.at[wid], dsts_v, sem1).wait()
    plsc.subcore_barrier()

    pltpu.async_copy(y_hbm.at[sidx(0)], rows0_v, sem0)

    def pair(g, carry):
        j0 = g * 2
        j1 = j0 + 1
        pltpu.async_copy(y_hbm.at[sidx(j1)], rows1_v, sem1)
        pltpu.make_async_copy(y_hbm.at[sidx(j0)], rows0_v, sem0).wait()
        pltpu.sync_copy(rows0_v, acc_sh.at[dsts_v.at[j0]], add=True)
        pltpu.async_copy(y_hbm.at[sidx(j0 + 2)], rows0_v, sem0)
        pltpu.make_async_copy(y_hbm.at[sidx(j1)], rows1_v, sem1).wait()
        pltpu.sync_copy(rows1_v, acc_sh.at[dsts_v.at[j1]], add=True)
        return carry

    lax.fori_loop(0, NCH // 2, pair, 0)
    jl = NCH - 1
    pltpu.make_async_copy(y_hbm.at[sidx(jl)], rows0_v, sem0).wait()
    pltpu.sync_copy(rows0_v, acc_sh.at[dsts_v.at[jl]], add=True)

    plsc.subcore_barrier()

    def writeout(r0):
        sl = pl.ds(r0, RB)
        pltpu.sync_copy(acc_sh.at[sl, :], stage)
        pltpu.sync_copy(stage, out_hbm.at[c, sl, :])

    _row_blocks(s, writeout)


@functools.lru_cache(maxsize=None)
def _sc_kernels():
    mesh = plsc.VectorSubcoreMesh(
        core_axis_name="c", subcore_axis_name="s",
        num_cores=NC, num_subcores=NS)
    deg_kernel = pl.kernel(
        _deg_body,
        out_type=jax.ShapeDtypeStruct((NW * N,), jnp.float32),
        mesh=mesh,
        compiler_params=pltpu.CompilerParams(needs_layout_passes=False),
        scratch_types=[
            pltpu.VMEM((NCH, CH), jnp.int32),
            pltpu.VMEM((N,), jnp.float32),
            pltpu.SemaphoreType.DMA,
        ],
    )
    agg_kernel = pl.kernel(
        _agg_body,
        out_type=jax.ShapeDtypeStruct((NC, N, H), jnp.float32),
        mesh=mesh,
        scratch_types=[
            pltpu.VMEM((EW,), jnp.int32),
            pltpu.VMEM((NCH, CH), jnp.int32),
            pltpu.VMEM((CH, H), jnp.float32),
            pltpu.VMEM((CH, H), jnp.float32),
            pltpu.VMEM_SHARED((N, H), jnp.float32),
            pltpu.SemaphoreType.DMA,
            pltpu.SemaphoreType.DMA,
        ],
    )
    return deg_kernel, agg_kernel


BLK = 1000
NBLK = N // BLK


def _dis_from(deg_ref3):
    deg_ref = deg_ref3[0]
    ones_col = jnp.ones((NW, 1), jnp.float32)
    d = lax.dot_general(deg_ref, ones_col, (((0,), (0,)), ((), ())),
                        preferred_element_type=jnp.float32,
                        precision=lax.Precision.HIGHEST) + 1.0
    return lax.rsqrt(d)


def _mm_scale_body(x_ref, w_ref, deg_ref, o_ref, disb_ref):
    dis = _dis_from(deg_ref)
    o_ref[...] = jnp.dot(x_ref[...], w_ref[...],
                         preferred_element_type=jnp.float32, precision=lax.Precision.HIGHEST) * dis
    disb_ref[...] = jnp.broadcast_to(dis, (BLK, H))


def _ln_relu(srow, g_ref, be_ref):
    mu = jnp.mean(srow, axis=1, keepdims=True)
    t = srow - mu
    var = jnp.mean(t * t, axis=1, keepdims=True)
    h = t * lax.rsqrt(var + 1e-5) * g_ref[...] + be_ref[...]
    return jnp.maximum(h, 0.0)


def _mid_body(agg_ref, y_ref, disb_ref, b_ref, g_ref, be_ref, w2_ref, o_ref):
    disb = disb_ref[...]
    srow = (agg_ref[0] + agg_ref[1] + y_ref[...]) * disb + b_ref[...]
    h = _ln_relu(srow, g_ref, be_ref)
    o_ref[...] = jnp.dot(h, w2_ref[...],
                         preferred_element_type=jnp.float32, precision=lax.Precision.HIGHEST) * disb


def _final_body(agg_ref, y_ref, disb_ref, b_ref, g_ref, be_ref, bt_ref,
                wl_ref, bl_ref, o_ref, pool_acc, cnt_acc):
    i = pl.program_id(0)
    srow = (agg_ref[0] + agg_ref[1] + y_ref[...]) * disb_ref[...] + b_ref[...]
    h = _ln_relu(srow, g_ref, be_ref)
    onehot = (bt_ref[...] == lax.broadcasted_iota(jnp.int32, (1, 64), 1))
    p = onehot.astype(jnp.float32)
    dn = (((0,), (0,)), ((), ()))
    pp = lax.dot_general(p, h, dn, preferred_element_type=jnp.float32, precision=lax.Precision.HIGHEST)
    cc = lax.dot_general(p, jnp.ones_like(h), dn,
                         preferred_element_type=jnp.float32, precision=lax.Precision.HIGHEST)

    @pl.when(i == 0)
    def _():
        pool_acc[...] = pp
        cnt_acc[...] = cc

    @pl.when(i > 0)
    def _():
        pool_acc[...] += pp
        cnt_acc[...] += cc

    @pl.when(i == pl.num_programs(0) - 1)
    def _():
        pooled = pool_acc[...] / jnp.maximum(cnt_acc[...], 1.0)
        o_ref[...] = jnp.dot(pooled, wl_ref[...],
                             preferred_element_type=jnp.float32, precision=lax.Precision.HIGHEST) + bl_ref[...]


def _row_spec(last):
    return pl.BlockSpec((BLK, last), lambda i: (i, 0))


_FULL_W = pl.BlockSpec((H, H), lambda i: (0, 0))
_DEG_SPEC = pl.BlockSpec((1, NW, BLK), lambda i: (i, 0, 0))
_AGG_SPEC = pl.BlockSpec((NC, BLK, H), lambda i: (0, i, 0))
_VEC_SPEC = pl.BlockSpec((1, H), lambda i: (0, 0))

_mm_scale = pl.pallas_call(
    _mm_scale_body,
    grid=(NBLK,),
    in_specs=[_row_spec(H), _FULL_W, _DEG_SPEC],
    out_specs=[_row_spec(H), _row_spec(H)],
    out_shape=[jax.ShapeDtypeStruct((N, H), jnp.float32),
               jax.ShapeDtypeStruct((N, H), jnp.float32)],
)

_mid = pl.pallas_call(
    _mid_body,
    grid=(NBLK,),
    in_specs=[_AGG_SPEC, _row_spec(H), _row_spec(H),
              _VEC_SPEC, _VEC_SPEC, _VEC_SPEC, _FULL_W],
    out_specs=_row_spec(H),
    out_shape=jax.ShapeDtypeStruct((N, H), jnp.float32),
)

_final = pl.pallas_call(
    _final_body,
    grid=(NBLK,),
    in_specs=[_AGG_SPEC, _row_spec(H), _row_spec(H),
              _VEC_SPEC, _VEC_SPEC, _VEC_SPEC,
              pl.BlockSpec((BLK, 1), lambda i: (i, 0)),
              pl.BlockSpec((H, 10), lambda i: (0, 0)),
              pl.BlockSpec((1, 10), lambda i: (0, 0))],
    out_specs=pl.BlockSpec((64, 10), lambda i: (0, 0)),
    out_shape=jax.ShapeDtypeStruct((64, 10), jnp.float32),
    scratch_shapes=[pltpu.VMEM((64, H), jnp.float32),
                    pltpu.VMEM((64, H), jnp.float32)],
)


def kernel(x, edge_index, batch, W1, b1, g1, be1, W2, b2, g2, be2, Wl, bl):
    src1 = edge_index[0]
    dst3 = edge_index[1].reshape(NW, NCH, CH)
    bt = batch.reshape(N, 1)

    deg_kernel, agg_kernel = _sc_kernels()
    deg = deg_kernel(dst3).reshape(NBLK, NW, BLK)
    y1, disb = _mm_scale(x, W1, deg)
    agg1 = agg_kernel(y1, src1, dst3)
    y2 = _mid(agg1, y1, disb, b1.reshape(1, H), g1.reshape(1, H),
              be1.reshape(1, H), W2)
    agg2 = agg_kernel(y2, src1, dst3)
    out = _final(agg2, y2, disb, b2.reshape(1, H), g2.reshape(1, H),
                 be2.reshape(1, H), bt, Wl, bl.reshape(1, 10))
    return out


# speedup vs baseline: 1.8449x; 1.0035x over previous
import functools

import jax
import jax.numpy as jnp
from jax import lax
from jax.experimental import pallas as pl
from jax.experimental.pallas import tpu as pltpu
from jax.experimental.pallas import tpu_sc as plsc

N = 10000
E = 320000
H = 128
NC = 2
NS = 16
NW = NC * NS
EW = E // NW
CH = 80
NCH = EW // CH
RB = 80
NRB = N // RB

def _row_blocks(s, fn):
    for j in range(8):
        blk = j * NS + s
        if j * NS + NS - 1 < NRB:
            fn(pl.multiple_of(blk * RB, 8))
        else:
            @pl.when(blk < NRB)
            def _():
                fn(pl.multiple_of(blk * RB, 8))


def _deg_body(dst3_hbm, out_hbm, didx_v, hist_v, dsem):
    c = lax.axis_index("c")
    s = lax.axis_index("s")
    wid = c * NS + s
    zero16 = jnp.zeros((16,), jnp.float32)
    idx_src = dst3_hbm.at[wid]
    pltpu.async_copy(idx_src, didx_v, dsem)

    def zstep(k, carry):
        hist_v[pl.ds(pl.multiple_of(k * 16, 8), 16)] = zero16
        return carry

    lax.fori_loop(0, N // 16, zstep, 0)
    pltpu.make_async_copy(idx_src, didx_v, dsem).wait()
    ones16 = jnp.ones((16,), jnp.float32)

    def step(j, carry):
        for k in range(CH // 16):
            iv = didx_v[j, pl.ds(k * 16, 16)]
            plsc.addupdate_scatter(hist_v, [iv], ones16)
        return carry

    lax.fori_loop(0, NCH, step, 0)
    for blk in range(NBLK):
        pltpu.sync_copy(
            hist_v.at[pl.ds(blk * BLK, BLK)],
            out_hbm.at[pl.ds(pl.multiple_of(blk * NW * BLK + wid * BLK, 8), BLK)])


def _agg_body(y_hbm, src_hbm, dst3_hbm, out_hbm,
              srcs_v, dsts_v, rows0_v, rows1_v, acc_sh, sem0, sem1):
    c = lax.axis_index("c")
    s = lax.axis_index("s")
    wid = c * NS + s
    src_src = src_hbm.at[pl.ds(pl.multiple_of(wid * EW, 8), EW)]
    pltpu.async_copy(src_src, srcs_v, sem0)
    pltpu.async_copy(dst3_hbm.at[wid], dsts_v, sem1)

    def sidx(j):
        return srcs_v.at[pl.ds(j * CH, CH)]
    stage = rows0_v.at[pl.ds(0, RB), :]
    zero16 = jnp.zeros((16,), jnp.float32)

    def zrow(r, carry):
        for cc in range(H // 16):
            rows0_v[r, pl.ds(cc * 16, 16)] = zero16
        return carry

    lax.fori_loop(0, RB, zrow, 0)
    _row_blocks(s, lambda r0: pltpu.sync_copy(stage, acc_sh.at[pl.ds(r0, RB), :]))
    pltpu.make_async_copy(src_src, srcs_v, sem0).wait()
    pltpu.make_async_copy(dst3_hbm.at[wid], dsts_v, sem1).wait()
    plsc.subcore_barrier()

    pltpu.async_copy(y_hbm.at[sidx(0)], rows0_v, sem0)

    def pair(g, carry):
        j0 = g * 2
        j1 = j0 + 1
        pltpu.async_copy(y_hbm.at[sidx(j1)], rows1_v, sem1)
        pltpu.make_async_copy(y_hbm.at[sidx(j0)], rows0_v, sem0).wait()
        pltpu.sync_copy(rows0_v, acc_sh.at[dsts_v.at[j0]], add=True)
        pltpu.async_copy(y_hbm.at[sidx(j0 + 2)], rows0_v, sem0)
        pltpu.make_async_copy(y_hbm.at[sidx(j1)], rows1_v, sem1).wait()
        pltpu.sync_copy(rows1_v, acc_sh.at[dsts_v.at[j1]], add=True)
        return carry

    lax.fori_loop(0, NCH // 2, pair, 0)
    jl = NCH - 1
    pltpu.make_async_copy(y_hbm.at[sidx(jl)], rows0_v, sem0).wait()
    pltpu.sync_copy(rows0_v, acc_sh.at[dsts_v.at[jl]], add=True)

    plsc.subcore_barrier()

    def writeout(r0):
        sl = pl.ds(r0, RB)
        pltpu.sync_copy(acc_sh.at[sl, :], stage)
        pltpu.sync_copy(stage, out_hbm.at[c, sl, :])

    _row_blocks(s, writeout)


@functools.lru_cache(maxsize=None)
def _sc_kernels():
    mesh = plsc.VectorSubcoreMesh(
        core_axis_name="c", subcore_axis_name="s",
        num_cores=NC, num_subcores=NS)
    deg_kernel = pl.kernel(
        _deg_body,
        out_type=jax.ShapeDtypeStruct((NW * N,), jnp.float32),
        mesh=mesh,
        compiler_params=pltpu.CompilerParams(needs_layout_passes=False),
        scratch_types=[
            pltpu.VMEM((NCH, CH), jnp.int32),
            pltpu.VMEM((N,), jnp.float32),
            pltpu.SemaphoreType.DMA,
        ],
    )
    agg_kernel = pl.kernel(
        _agg_body,
        out_type=jax.ShapeDtypeStruct((NC, N, H), jnp.float32),
        mesh=mesh,
        scratch_types=[
            pltpu.VMEM((EW,), jnp.int32),
            pltpu.VMEM((NCH, CH), jnp.int32),
            pltpu.VMEM((CH, H), jnp.float32),
            pltpu.VMEM((CH, H), jnp.float32),
            pltpu.VMEM_SHARED((N, H), jnp.float32),
            pltpu.SemaphoreType.DMA,
            pltpu.SemaphoreType.DMA,
        ],
    )
    return deg_kernel, agg_kernel


BLK = 1000
NBLK = N // BLK


def _dis_from(deg_ref3):
    deg_ref = deg_ref3[0]
    ones_col = jnp.ones((NW, 1), jnp.float32)
    d = lax.dot_general(deg_ref, ones_col, (((0,), (0,)), ((), ())),
                        preferred_element_type=jnp.float32,
                        precision=lax.Precision.HIGHEST) + 1.0
    return lax.rsqrt(d)


def _mm_scale_body(x_ref, w_ref, deg_ref, o_ref, disb_ref):
    dis = _dis_from(deg_ref)
    o_ref[...] = jnp.dot(x_ref[...], w_ref[...],
                         preferred_element_type=jnp.float32, precision=lax.Precision.HIGHEST) * dis
    disb_ref[...] = jnp.broadcast_to(dis, (BLK, H))


def _ln_relu(srow, g_ref, be_ref):
    mu = jnp.mean(srow, axis=1, keepdims=True)
    t = srow - mu
    var = jnp.mean(t * t, axis=1, keepdims=True)
    h = t * lax.rsqrt(var + 1e-5) * g_ref[...] + be_ref[...]
    return jnp.maximum(h, 0.0)


def _mid_body(agg_ref, y_ref, disb_ref, b_ref, g_ref, be_ref, w2_ref, o_ref):
    disb = disb_ref[...]
    srow = (agg_ref[0] + agg_ref[1] + y_ref[...]) * disb + b_ref[...]
    h = _ln_relu(srow, g_ref, be_ref)
    o_ref[...] = jnp.dot(h, w2_ref[...],
                         preferred_element_type=jnp.float32, precision=lax.Precision.HIGHEST) * disb


def _final_body(agg_ref, y_ref, disb_ref, b_ref, g_ref, be_ref, bt_ref,
                wl_ref, bl_ref, o_ref, pool_acc, cnt_acc):
    i = pl.program_id(0)
    srow = (agg_ref[0] + agg_ref[1] + y_ref[...]) * disb_ref[...] + b_ref[...]
    h = _ln_relu(srow, g_ref, be_ref)
    p = bt_ref[...]
    dn = (((0,), (0,)), ((), ()))
    pp = lax.dot_general(p, h, dn, preferred_element_type=jnp.float32, precision=lax.Precision.HIGHEST)
    cc = lax.dot_general(p, jnp.ones_like(h), dn,
                         preferred_element_type=jnp.float32, precision=lax.Precision.HIGHEST)

    @pl.when(i == 0)
    def _():
        pool_acc[...] = pp
        cnt_acc[...] = cc

    @pl.when(i > 0)
    def _():
        pool_acc[...] += pp
        cnt_acc[...] += cc

    @pl.when(i == pl.num_programs(0) - 1)
    def _():
        pooled = pool_acc[...] / jnp.maximum(cnt_acc[...], 1.0)
        o_ref[...] = jnp.dot(pooled, wl_ref[...],
                             preferred_element_type=jnp.float32, precision=lax.Precision.HIGHEST) + bl_ref[...]


def _row_spec(last):
    return pl.BlockSpec((BLK, last), lambda i: (i, 0))


_FULL_W = pl.BlockSpec((H, H), lambda i: (0, 0))
_DEG_SPEC = pl.BlockSpec((1, NW, BLK), lambda i: (i, 0, 0))
_AGG_SPEC = pl.BlockSpec((NC, BLK, H), lambda i: (0, i, 0))
_VEC_SPEC = pl.BlockSpec((1, H), lambda i: (0, 0))

_mm_scale = pl.pallas_call(
    _mm_scale_body,
    grid=(NBLK,),
    in_specs=[_row_spec(H), _FULL_W, _DEG_SPEC],
    out_specs=[_row_spec(H), _row_spec(H)],
    out_shape=[jax.ShapeDtypeStruct((N, H), jnp.float32),
               jax.ShapeDtypeStruct((N, H), jnp.float32)],
)

_mid = pl.pallas_call(
    _mid_body,
    grid=(NBLK,),
    in_specs=[_AGG_SPEC, _row_spec(H), _row_spec(H),
              _VEC_SPEC, _VEC_SPEC, _VEC_SPEC, _FULL_W],
    out_specs=_row_spec(H),
    out_shape=jax.ShapeDtypeStruct((N, H), jnp.float32),
)

_final = pl.pallas_call(
    _final_body,
    grid=(NBLK,),
    in_specs=[_AGG_SPEC, _row_spec(H), _row_spec(H),
              _VEC_SPEC, _VEC_SPEC, _VEC_SPEC,
              pl.BlockSpec((BLK, 64), lambda i: (i, 0)),
              pl.BlockSpec((H, 10), lambda i: (0, 0)),
              pl.BlockSpec((1, 10), lambda i: (0, 0))],
    out_specs=pl.BlockSpec((64, 10), lambda i: (0, 0)),
    out_shape=jax.ShapeDtypeStruct((64, 10), jnp.float32),
    scratch_shapes=[pltpu.VMEM((64, H), jnp.float32),
                    pltpu.VMEM((64, H), jnp.float32)],
)


def kernel(x, edge_index, batch, W1, b1, g1, be1, W2, b2, g2, be2, Wl, bl):
    src1 = edge_index[0]
    dst3 = edge_index[1].reshape(NW, NCH, CH)
    bt = (batch[:, None] == jnp.arange(64, dtype=batch.dtype)[None, :]
          ).astype(jnp.float32)

    deg_kernel, agg_kernel = _sc_kernels()
    deg = deg_kernel(dst3).reshape(NBLK, NW, BLK)
    y1, disb = _mm_scale(x, W1, deg)
    agg1 = agg_kernel(y1, src1, dst3)
    y2 = _mid(agg1, y1, disb, b1.reshape(1, H), g1.reshape(1, H),
              be1.reshape(1, H), W2)
    agg2 = agg_kernel(y2, src1, dst3)
    out = _final(agg2, y2, disb, b2.reshape(1, H), g2.reshape(1, H),
                 be2.reshape(1, H), bt, Wl, bl.reshape(1, 10))
    return out


# speedup vs baseline: 1.9264x; 1.0442x over previous
import functools

import jax
import jax.numpy as jnp
from jax import lax
from jax.experimental import pallas as pl
from jax.experimental.pallas import tpu as pltpu
from jax.experimental.pallas import tpu_sc as plsc

N = 10000
E = 320000
H = 128
NC = 2
NS = 16
NW = NC * NS
EW = E // NW
CH = 80
NCH = EW // CH
RB = 80
NRB = N // RB

def _row_blocks(s, fn):
    for j in range(8):
        blk = j * NS + s
        if j * NS + NS - 1 < NRB:
            fn(pl.multiple_of(blk * RB, 8))
        else:
            @pl.when(blk < NRB)
            def _():
                fn(pl.multiple_of(blk * RB, 8))


def _deg_body(dst3_hbm, out_hbm, didx_v, hist_v, dsem):
    c = lax.axis_index("c")
    s = lax.axis_index("s")
    wid = c * NS + s
    zero16 = jnp.zeros((16,), jnp.float32)
    idx_src = dst3_hbm.at[wid]
    pltpu.async_copy(idx_src, didx_v, dsem)

    def zstep(k, carry):
        hist_v[pl.ds(pl.multiple_of(k * 16, 8), 16)] = zero16
        return carry

    lax.fori_loop(0, N // 16, zstep, 0)
    pltpu.make_async_copy(idx_src, didx_v, dsem).wait()
    ones16 = jnp.ones((16,), jnp.float32)

    def step(j, carry):
        for k in range(CH // 16):
            iv = didx_v[j, pl.ds(k * 16, 16)]
            plsc.addupdate_scatter(hist_v, [iv], ones16)
        return carry

    lax.fori_loop(0, NCH, step, 0)
    for blk in range(NBLK):
        pltpu.sync_copy(
            hist_v.at[pl.ds(blk * BLK, BLK)],
            out_hbm.at[pl.ds(pl.multiple_of(blk * NW * BLK + wid * BLK, 8), BLK)])


def _agg_body(y_hbm, src_hbm, dst3_hbm, out_hbm,
              srcs_v, dsts_v, rows0_v, rows1_v, acc_sh, sem0, sem1):
    c = lax.axis_index("c")
    s = lax.axis_index("s")
    wid = c * NS + s
    src_src = src_hbm.at[pl.ds(pl.multiple_of(wid * EW, 8), EW)]
    pltpu.async_copy(src_src, srcs_v, sem0)
    pltpu.async_copy(dst3_hbm.at[wid], dsts_v, sem1)

    def sidx(j):
        return srcs_v.at[pl.ds(j * CH, CH)]
    stage = rows0_v.at[pl.ds(0, RB), :]
    zero16 = jnp.zeros((16,), jnp.float32)

    def zrow(r, carry):
        for cc in range(H // 16):
            rows0_v[r, pl.ds(cc * 16, 16)] = zero16
        return carry

    lax.fori_loop(0, RB, zrow, 0)
    _row_blocks(s, lambda r0: pltpu.sync_copy(stage, acc_sh.at[pl.ds(r0, RB), :]))
    pltpu.make_async_copy(src_src, srcs_v, sem0).wait()
    pltpu.make_async_copy(dst3_hbm.at[wid], dsts_v, sem1).wait()
    plsc.subcore_barrier()

    pltpu.async_copy(y_hbm.at[sidx(0)], rows0_v, sem0)

    def pair(g, carry):
        j0 = g * 2
        j1 = j0 + 1
        pltpu.async_copy(y_hbm.at[sidx(j1)], rows1_v, sem1)
        pltpu.make_async_copy(y_hbm.at[sidx(j0)], rows0_v, sem0).wait()
        pltpu.sync_copy(rows0_v, acc_sh.at[dsts_v.at[j0]], add=True)
        pltpu.async_copy(y_hbm.at[sidx(j0 + 2)], rows0_v, sem0)
        pltpu.make_async_copy(y_hbm.at[sidx(j1)], rows1_v, sem1).wait()
        pltpu.sync_copy(rows1_v, acc_sh.at[dsts_v.at[j1]], add=True)
        return carry

    lax.fori_loop(0, NCH // 2, pair, 0)
    jl = NCH - 1
    pltpu.make_async_copy(y_hbm.at[sidx(jl)], rows0_v, sem0).wait()
    pltpu.sync_copy(rows0_v, acc_sh.at[dsts_v.at[jl]], add=True)

    plsc.subcore_barrier()

    def writeout(r0):
        sl = pl.ds(r0, RB)
        pltpu.sync_copy(acc_sh.at[sl, :], stage)
        pltpu.sync_copy(stage, out_hbm.at[c, sl, :])

    _row_blocks(s, writeout)


@functools.lru_cache(maxsize=None)
def _sc_kernels():
    mesh = plsc.VectorSubcoreMesh(
        core_axis_name="c", subcore_axis_name="s",
        num_cores=NC, num_subcores=NS)
    deg_kernel = pl.kernel(
        _deg_body,
        out_type=jax.ShapeDtypeStruct((NW * N,), jnp.float32),
        mesh=mesh,
        compiler_params=pltpu.CompilerParams(needs_layout_passes=False),
        scratch_types=[
            pltpu.VMEM((NCH, CH), jnp.int32),
            pltpu.VMEM((N,), jnp.float32),
            pltpu.SemaphoreType.DMA,
        ],
    )
    agg_kernel = pl.kernel(
        _agg_body,
        out_type=jax.ShapeDtypeStruct((NC, N, H), jnp.float32),
        mesh=mesh,
        scratch_types=[
            pltpu.VMEM((EW,), jnp.int32),
            pltpu.VMEM((NCH, CH), jnp.int32),
            pltpu.VMEM((CH, H), jnp.float32),
            pltpu.VMEM((CH, H), jnp.float32),
            pltpu.VMEM_SHARED((N, H), jnp.float32),
            pltpu.SemaphoreType.DMA,
            pltpu.SemaphoreType.DMA,
        ],
    )
    return deg_kernel, agg_kernel


BLK = 2000
NBLK = N // BLK


def _dis_from(deg_ref3):
    deg_ref = deg_ref3[0]
    ones_col = jnp.ones((NW, 1), jnp.float32)
    d = lax.dot_general(deg_ref, ones_col, (((0,), (0,)), ((), ())),
                        preferred_element_type=jnp.float32,
                        precision=lax.Precision.HIGHEST) + 1.0
    return lax.rsqrt(d)


def _mm_scale_body(x_ref, w_ref, deg_ref, o_ref, disb_ref):
    dis = _dis_from(deg_ref)
    o_ref[...] = jnp.dot(x_ref[...], w_ref[...],
                         preferred_element_type=jnp.float32, precision=lax.Precision.HIGHEST) * dis
    disb_ref[...] = jnp.broadcast_to(dis, (BLK, H))


def _ln_relu(srow, g_ref, be_ref):
    mu = jnp.mean(srow, axis=1, keepdims=True)
    t = srow - mu
    var = jnp.mean(t * t, axis=1, keepdims=True)
    h = t * lax.rsqrt(var + 1e-5) * g_ref[...] + be_ref[...]
    return jnp.maximum(h, 0.0)


def _mid_body(agg_ref, y_ref, disb_ref, b_ref, g_ref, be_ref, w2_ref, o_ref):
    disb = disb_ref[...]
    srow = (agg_ref[0] + agg_ref[1] + y_ref[...]) * disb + b_ref[...]
    h = _ln_relu(srow, g_ref, be_ref)
    o_ref[...] = jnp.dot(h, w2_ref[...],
                         preferred_element_type=jnp.float32, precision=lax.Precision.HIGHEST) * disb


def _final_body(agg_ref, y_ref, disb_ref, b_ref, g_ref, be_ref, bt_ref,
                wl_ref, bl_ref, o_ref, pool_acc, cnt_acc):
    i = pl.program_id(0)
    srow = (agg_ref[0] + agg_ref[1] + y_ref[...]) * disb_ref[...] + b_ref[...]
    h = _ln_relu(srow, g_ref, be_ref)
    p = bt_ref[...]
    dn = (((0,), (0,)), ((), ()))
    pp = lax.dot_general(p, h, dn, preferred_element_type=jnp.float32, precision=lax.Precision.HIGHEST)
    cc = lax.dot_general(p, jnp.ones_like(h), dn,
                         preferred_element_type=jnp.float32, precision=lax.Precision.HIGHEST)

    @pl.when(i == 0)
    def _():
        pool_acc[...] = pp
        cnt_acc[...] = cc

    @pl.when(i > 0)
    def _():
        pool_acc[...] += pp
        cnt_acc[...] += cc

    @pl.when(i == pl.num_programs(0) - 1)
    def _():
        pooled = pool_acc[...] / jnp.maximum(cnt_acc[...], 1.0)
        o_ref[...] = jnp.dot(pooled, wl_ref[...],
                             preferred_element_type=jnp.float32, precision=lax.Precision.HIGHEST) + bl_ref[...]


def _row_spec(last):
    return pl.BlockSpec((BLK, last), lambda i: (i, 0))


_FULL_W = pl.BlockSpec((H, H), lambda i: (0, 0))
_DEG_SPEC = pl.BlockSpec((1, NW, BLK), lambda i: (i, 0, 0))
_AGG_SPEC = pl.BlockSpec((NC, BLK, H), lambda i: (0, i, 0))
_VEC_SPEC = pl.BlockSpec((1, H), lambda i: (0, 0))

_mm_scale = pl.pallas_call(
    _mm_scale_body,
    grid=(NBLK,),
    in_specs=[_row_spec(H), _FULL_W, _DEG_SPEC],
    out_specs=[_row_spec(H), _row_spec(H)],
    out_shape=[jax.ShapeDtypeStruct((N, H), jnp.float32),
               jax.ShapeDtypeStruct((N, H), jnp.float32)],
)

_mid = pl.pallas_call(
    _mid_body,
    grid=(NBLK,),
    in_specs=[_AGG_SPEC, _row_spec(H), _row_spec(H),
              _VEC_SPEC, _VEC_SPEC, _VEC_SPEC, _FULL_W],
    out_specs=_row_spec(H),
    out_shape=jax.ShapeDtypeStruct((N, H), jnp.float32),
)

_final = pl.pallas_call(
    _final_body,
    grid=(NBLK,),
    in_specs=[_AGG_SPEC, _row_spec(H), _row_spec(H),
              _VEC_SPEC, _VEC_SPEC, _VEC_SPEC,
              pl.BlockSpec((BLK, 64), lambda i: (i, 0)),
              pl.BlockSpec((H, 10), lambda i: (0, 0)),
              pl.BlockSpec((1, 10), lambda i: (0, 0))],
    out_specs=pl.BlockSpec((64, 10), lambda i: (0, 0)),
    out_shape=jax.ShapeDtypeStruct((64, 10), jnp.float32),
    scratch_shapes=[pltpu.VMEM((64, H), jnp.float32),
                    pltpu.VMEM((64, H), jnp.float32)],
)


def kernel(x, edge_index, batch, W1, b1, g1, be1, W2, b2, g2, be2, Wl, bl):
    src1 = edge_index[0]
    dst3 = edge_index[1].reshape(NW, NCH, CH)
    bt = (batch[:, None] == jnp.arange(64, dtype=batch.dtype)[None, :]
          ).astype(jnp.float32)

    deg_kernel, agg_kernel = _sc_kernels()
    deg = deg_kernel(dst3).reshape(NBLK, NW, BLK)
    y1, disb = _mm_scale(x, W1, deg)
    agg1 = agg_kernel(y1, src1, dst3)
    y2 = _mid(agg1, y1, disb, b1.reshape(1, H), g1.reshape(1, H),
              be1.reshape(1, H), W2)
    agg2 = agg_kernel(y2, src1, dst3)
    out = _final(agg2, y2, disb, b2.reshape(1, H), g2.reshape(1, H),
                 be2.reshape(1, H), bt, Wl, bl.reshape(1, 10))
    return out
